# baseline (speedup 1.0000x reference)
import jax
import jax.numpy as jnp
from jax.experimental import pallas as pl
from jax.experimental.pallas import tpu as pltpu

MIN_S, MAX_S = 0.05, 0.3

_K_BLK = 1024
_ROW_BLK = 128


def _predictor_kernel(x_ref, w1_ref, b1_ref, w2_ref, b2_ref,
                      sp_ref, k_ref, acc_ref):
    j = pl.program_id(0)
    d = pl.num_programs(0) * x_ref.shape[1]
    part = jnp.dot(x_ref[...], w1_ref[...], preferred_element_type=jnp.float32)

    @pl.when(j == 0)
    def _():
        acc_ref[...] = part

    @pl.when(j > 0)
    def _():
        acc_ref[...] += part

    @pl.when(j == pl.num_programs(0) - 1)
    def _():
        h = jnp.maximum(acc_ref[...] + b1_ref[...], 0.0)
        logit = jnp.dot(h, w2_ref[...], preferred_element_type=jnp.float32)
        s = jax.nn.sigmoid(logit + b2_ref[...])
        sp = MIN_S + (MAX_S - MIN_S) * s
        sp_ref[...] = sp
        k = jnp.round(jnp.float32(d) * (1.0 - sp)).astype(jnp.int32)
        k_ref[...] = jnp.clip(k, 1, d)


def _select_kernel(x_ref, k_ref, sx_ref, mask_ref, asp_ref, l1_ref):
    i = pl.program_id(0)
    x = x_ref[...]
    rb, d = x.shape
    ax = jnp.bitwise_and(jax.lax.bitcast_convert_type(x, jnp.int32),
                         jnp.int32(0x7FFFFFFF))
    k = k_ref[...]

    def body(_, carry):
        lo, hi = carry
        span = hi - lo
        m1 = lo + jax.lax.shift_right_logical(span, 2)
        m2 = lo + jax.lax.shift_right_logical(span, 1)
        m3 = hi - jax.lax.shift_right_logical(span, 2)
        c1 = jnp.sum((ax <= m1).astype(jnp.int32), axis=1, keepdims=True)
        c2 = jnp.sum((ax <= m2).astype(jnp.int32), axis=1, keepdims=True)
        c3 = jnp.sum((ax <= m3).astype(jnp.int32), axis=1, keepdims=True)
        g1, g2, g3 = c1 >= k, c2 >= k, c3 >= k
        lo = jnp.where(g1, lo,
                       jnp.where(g2, m1 + 1, jnp.where(g3, m2 + 1, m3 + 1)))
        hi = jnp.where(g1, m1, jnp.where(g2, m2, jnp.where(g3, m3, hi)))
        return lo, hi

    lo0 = jnp.zeros_like(k)
    hi0 = jnp.full_like(k, jnp.int32(0x7F800000))
    thr, _ = jax.lax.fori_loop(0, 16, body, (lo0, hi0))

    maskf = (ax > thr).astype(jnp.float32)
    sx = x * maskf
    sx_ref[...] = sx
    mask_ref[...] = maskf
    asp_ref[...] = jnp.sum(maskf, axis=1, keepdims=True) * (1.0 / d)
    part = (jnp.sum(jnp.abs(sx)) * (1.0 / (rb * pl.num_programs(0)))
            ).reshape(1, 1)

    @pl.when(i == 0)
    def _():
        l1_ref[...] = part

    @pl.when(i > 0)
    def _():
        l1_ref[...] += part


def kernel(x, W1, b1, W2, b2):
    B, D = x.shape
    H = W1.shape[1]
    nk = D // _K_BLK

    sparsity, k = pl.pallas_call(
        _predictor_kernel,
        grid=(nk,),
        in_specs=[
            pl.BlockSpec((B, _K_BLK), lambda j: (0, j)),
            pl.BlockSpec((_K_BLK, H), lambda j: (j, 0)),
            pl.BlockSpec((1, H), lambda j: (0, 0)),
            pl.BlockSpec((H, 1), lambda j: (0, 0)),
            pl.BlockSpec((1, 1), lambda j: (0, 0)),
        ],
        out_specs=[
            pl.BlockSpec((B, 1), lambda j: (0, 0)),
            pl.BlockSpec((B, 1), lambda j: (0, 0)),
        ],
        out_shape=[
            jax.ShapeDtypeStruct((B, 1), jnp.float32),
            jax.ShapeDtypeStruct((B, 1), jnp.int32),
        ],
        scratch_shapes=[pltpu.VMEM((B, H), jnp.float32)],
    )(x, W1, b1.reshape(1, H), W2, b2.reshape(1, 1))

    nrows = B // _ROW_BLK
    sparse_x, mask, asp, l1 = pl.pallas_call(
        _select_kernel,
        grid=(nrows,),
        in_specs=[
            pl.BlockSpec((_ROW_BLK, D), lambda i: (i, 0)),
            pl.BlockSpec((_ROW_BLK, 1), lambda i: (i, 0)),
        ],
        out_specs=[
            pl.BlockSpec((_ROW_BLK, D), lambda i: (i, 0)),
            pl.BlockSpec((_ROW_BLK, D), lambda i: (i, 0)),
            pl.BlockSpec((_ROW_BLK, 1), lambda i: (i, 0)),
            pl.BlockSpec((1, 1), lambda i: (0, 0)),
        ],
        out_shape=[
            jax.ShapeDtypeStruct((B, D), jnp.float32),
            jax.ShapeDtypeStruct((B, D), jnp.float32),
            jax.ShapeDtypeStruct((B, 1), jnp.float32),
            jax.ShapeDtypeStruct((1, 1), jnp.float32),
        ],
    )(x, k)

    return (sparse_x, mask, sparsity, asp.reshape(B), l1.reshape(()))


# speedup vs baseline: 1.0317x; 1.0317x over previous
import jax
import jax.numpy as jnp
from jax.experimental import pallas as pl
from jax.experimental.pallas import tpu as pltpu

MIN_S, MAX_S = 0.05, 0.3

_K_BLK = 1024


def _fused_kernel(x_ref, w1_ref, b1_ref, w2_ref, b2_ref,
                  sp_ref, sx_ref, mask_ref, asp_ref, l1_ref,
                  acc_ref, k_ref):
    j = pl.program_id(0)
    nk = pl.num_programs(0) - 1

    @pl.when(j < nk)
    def _matmul_step():
        xs = x_ref[:, pl.ds(j * _K_BLK, _K_BLK)]
        part = jnp.dot(xs, w1_ref[...], preferred_element_type=jnp.float32)

        @pl.when(j == 0)
        def _():
            acc_ref[...] = part

        @pl.when(j > 0)
        def _():
            acc_ref[...] += part

    @pl.when(j == nk - 1)
    def _head_step():
        d = x_ref.shape[1]
        h = jnp.maximum(acc_ref[...] + b1_ref[...], 0.0)
        logit = jnp.dot(h, w2_ref[...], preferred_element_type=jnp.float32)
        s = jax.nn.sigmoid(logit + b2_ref[...])
        sp = MIN_S + (MAX_S - MIN_S) * s
        sp_ref[...] = sp
        k = jnp.round(jnp.float32(d) * (1.0 - sp)).astype(jnp.int32)
        k_ref[...] = jnp.clip(k, 1, d)

    @pl.when(j == nk)
    def _select_step():
        x = x_ref[...]
        rb, d = x.shape
        ax = jnp.bitwise_and(jax.lax.bitcast_convert_type(x, jnp.int32),
                             jnp.int32(0x7FFFFFFF))
        k = k_ref[...]

        def body(_, carry):
            lo, hi, c = carry
            mid = lo + jax.lax.shift_right_logical(hi - lo, 1)
            cnt = jnp.sum((ax <= mid).astype(jnp.int32), axis=1,
                          keepdims=True)
            ge = cnt >= k
            return (jnp.where(ge, lo, mid + 1), jnp.where(ge, mid, hi),
                    jnp.where(ge, cnt, c))

        lo0 = jnp.zeros_like(k)
        hi0 = jnp.full_like(k, jnp.int32(0x7F800000))
        c0 = jnp.full_like(k, d)
        thr, _, cthr = jax.lax.fori_loop(0, 31, body, (lo0, hi0, c0))

        maskf = (ax > thr).astype(jnp.float32)
        sx = x * maskf
        sx_ref[...] = sx
        mask_ref[...] = maskf
        asp_ref[...] = (d - cthr).astype(jnp.float32) * (1.0 / d)
        l1_ref[...] = (jnp.sum(jnp.abs(sx)) * (1.0 / rb)).reshape(1, 1)


def kernel(x, W1, b1, W2, b2):
    B, D = x.shape
    H = W1.shape[1]
    nk = D // _K_BLK

    sparsity, sparse_x, mask, asp, l1 = pl.pallas_call(
        _fused_kernel,
        grid=(nk + 1,),
        in_specs=[
            pl.BlockSpec((B, D), lambda j: (0, 0)),
            pl.BlockSpec((_K_BLK, H), lambda j: (jnp.minimum(j, D // _K_BLK - 1), 0)),
            pl.BlockSpec((1, H), lambda j: (0, 0)),
            pl.BlockSpec((H, 1), lambda j: (0, 0)),
            pl.BlockSpec((1, 1), lambda j: (0, 0)),
        ],
        out_specs=[
            pl.BlockSpec((B, 1), lambda j: (0, 0)),
            pl.BlockSpec((B, D), lambda j: (0, 0)),
            pl.BlockSpec((B, D), lambda j: (0, 0)),
            pl.BlockSpec((B, 1), lambda j: (0, 0)),
            pl.BlockSpec((1, 1), lambda j: (0, 0)),
        ],
        out_shape=[
            jax.ShapeDtypeStruct((B, 1), jnp.float32),
            jax.ShapeDtypeStruct((B, D), jnp.float32),
            jax.ShapeDtypeStruct((B, D), jnp.float32),
            jax.ShapeDtypeStruct((B, 1), jnp.float32),
            jax.ShapeDtypeStruct((1, 1), jnp.float32),
        ],
        scratch_shapes=[
            pltpu.VMEM((B, H), jnp.float32),
            pltpu.VMEM((B, 1), jnp.int32),
        ],
    )(x, W1, b1.reshape(1, H), W2, b2.reshape(1, 1))

    return (sparse_x, mask, sparsity, asp.reshape(B), l1.reshape(()))
